# Initial kernel scaffold; baseline (speedup 1.0000x reference)
import jax
import jax.numpy as jnp
from jax.experimental import pallas as pl


def kernel(x, Wq, Wk, Wg, ape):
    raise NotImplementedError("write your pallas kernel here")


# speedup vs baseline: 5.4918x; 5.4918x over previous
import functools
import math

import jax
import jax.numpy as jnp
from jax.experimental import pallas as pl

H = 16
D = 64
HD = H * D
R = 4
EPS = 1e-6


def _head_block(dtype=jnp.float32):
    i = jax.lax.broadcasted_iota(jnp.int32, (HD, HD), 0) // D
    j = jax.lax.broadcasted_iota(jnp.int32, (HD, HD), 1) // D
    return (i == j).astype(dtype)


def _keys_kernel(x_ref, wk_ref, wg_ref, ape_ref, out_ref, *, bg):
    x = x_ref[0].astype(jnp.bfloat16)
    k = jnp.dot(x, wk_ref[...].astype(jnp.bfloat16), preferred_element_type=jnp.float32)
    g = jnp.dot(x, wg_ref[...].astype(jnp.bfloat16), preferred_element_type=jnp.float32)
    g = g + ape_ref[...]
    g = g - jnp.max(g)
    e = jnp.exp(g)
    rows = R * bg
    seg = (jax.lax.broadcasted_iota(jnp.int32, (bg, rows), 0)
           == jax.lax.broadcasted_iota(jnp.int32, (bg, rows), 1) // R)
    segf = seg.astype(jnp.float32)
    denom = jnp.dot(segf, e, preferred_element_type=jnp.float32, precision=jax.lax.Precision.HIGHEST)
    num = jnp.dot(segf, e * k, preferred_element_type=jnp.float32, precision=jax.lax.Precision.HIGHEST)
    keys = num / denom
    ssq = jnp.dot(keys * keys, _head_block(), preferred_element_type=jnp.float32, precision=jax.lax.Precision.HIGHEST)
    out_ref[0] = keys * jax.lax.rsqrt(ssq * (1.0 / D) + EPS)


def _scores_kernel(x_ref, wq_ref, keys_ref, out_ref, *, bt, g_tot, topk):
    tb = pl.program_id(1)
    x = x_ref[0].astype(jnp.bfloat16)
    q = jnp.dot(x, wq_ref[...].astype(jnp.bfloat16), preferred_element_type=jnp.float32)
    ssq = jnp.dot(q * q, _head_block(), preferred_element_type=jnp.float32, precision=jax.lax.Precision.HIGHEST)
    qn = q * jax.lax.rsqrt(ssq * (1.0 / D) + EPS)
    keys = keys_ref[0].astype(jnp.bfloat16)
    s = jax.lax.dot_general(qn.astype(jnp.bfloat16), keys, (((1,), (1,)), ((), ())),
                            preferred_element_type=jnp.float32)
    s = s * (1.0 / (H * math.sqrt(D)))
    t_idx = tb * bt + jax.lax.broadcasted_iota(jnp.int32, (bt, g_tot), 0)
    g_end = R * jax.lax.broadcasted_iota(jnp.int32, (bt, g_tot), 1) + (R - 1)
    causal = g_end <= t_idx
    neg = jnp.float32(-jnp.inf)
    s = jnp.where(causal, s, neg)
    cur = s
    th = None
    for _ in range(topk):
        th = jnp.max(cur, axis=1, keepdims=True)
        cur = jnp.where(cur >= th, neg, cur)
    out_ref[0] = (causal & (s >= th)).astype(jnp.int8)


def kernel(x, Wq, Wk, Wg, ape):
    B, T, E = x.shape
    G = T // R
    BG = min(128, G)
    BT = min(256, T)
    ape2 = jnp.tile(ape.reshape(R, HD), (BG, 1))
    keys = pl.pallas_call(
        functools.partial(_keys_kernel, bg=BG),
        grid=(B, G // BG),
        in_specs=[
            pl.BlockSpec((1, R * BG, E), lambda b, gb: (b, gb, 0)),
            pl.BlockSpec((E, HD), lambda b, gb: (0, 0)),
            pl.BlockSpec((E, HD), lambda b, gb: (0, 0)),
            pl.BlockSpec((R * BG, HD), lambda b, gb: (0, 0)),
        ],
        out_specs=pl.BlockSpec((1, BG, HD), lambda b, gb: (b, gb, 0)),
        out_shape=jax.ShapeDtypeStruct((B, G, HD), jnp.float32),
    )(x, Wk, Wg, ape2)
    topk = min(8, G)
    mask8 = pl.pallas_call(
        functools.partial(_scores_kernel, bt=BT, g_tot=G, topk=topk),
        grid=(B, T // BT),
        in_specs=[
            pl.BlockSpec((1, BT, E), lambda b, tb: (b, tb, 0)),
            pl.BlockSpec((E, HD), lambda b, tb: (0, 0)),
            pl.BlockSpec((1, G, HD), lambda b, tb: (b, 0, 0)),
        ],
        out_specs=pl.BlockSpec((1, BT, G), lambda b, tb: (b, tb, 0)),
        out_shape=jax.ShapeDtypeStruct((B, T, G), jnp.int8),
    )(x, Wq, keys)
    group_ends = jnp.minimum(jnp.arange(R - 1, G * R, R), T - 1)
    return mask8.astype(bool), group_ends
